# baseline (speedup 1.0000x reference)
import jax
import jax.numpy as jnp
from jax.experimental import pallas as pl
from jax.experimental.pallas import tpu as pltpu

IN_F = 1024
TS = 512
R = 8
SLAB = TS * R


def _make_body(out_f, nslab, last_rows, grid_s):
    def body(x_ref, b_ref, w_hbm, o_ref, w_buf, sem):
        s = pl.program_id(0)
        j = s // R
        i = s % R
        k = j % 2

        def full_copy(jn, kn):
            return pltpu.make_async_copy(
                w_hbm.at[pl.ds(jn * SLAB, SLAB), :], w_buf.at[kn],
                sem.at[kn])

        def short_copy(jn, kn):
            return pltpu.make_async_copy(
                w_hbm.at[pl.ds(jn * SLAB, last_rows), :],
                w_buf.at[kn, pl.ds(0, last_rows), :], sem.at[kn])

        def start_slab(jn, kn):
            @pl.when(jn < nslab - 1)
            def _():
                full_copy(jn, kn).start()

            @pl.when(jn == nslab - 1)
            def _():
                short_copy(jn, kn).start()

        @pl.when(i == 0)
        def _fetch():
            @pl.when(s == 0)
            def _prologue():
                start_slab(0, 0)

            @pl.when(j + 1 < nslab)
            def _next():
                start_slab(j + 1, (j + 1) % 2)

            @pl.when(j < nslab - 1)
            def _():
                full_copy(j, k).wait()

            @pl.when(j == nslab - 1)
            def _():
                short_copy(j, k).wait()

        w = w_buf[k, pl.ds(i * TS, TS), :]
        acc = jax.lax.dot_general(
            x_ref[...], w,
            dimension_numbers=(((1,), (1,)), ((), ())),
            preferred_element_type=jnp.float32,
            precision=jax.lax.Precision.DEFAULT,
        )
        o_ref[...] = acc + b_ref[:, pl.ds(s * TS, TS)]

    return body


def kernel(x, W, b):
    batch, in_f = x.shape
    out_f = W.shape[0]
    grid_s = (out_f + TS - 1) // TS
    nslab = (out_f + SLAB - 1) // SLAB
    last_rows = out_f - (nslab - 1) * SLAB
    b2 = jnp.pad(b, (0, grid_s * TS - out_f)).reshape(1, grid_s * TS)
    return pl.pallas_call(
        _make_body(out_f, nslab, last_rows, grid_s),
        grid=(grid_s,),
        in_specs=[
            pl.BlockSpec((batch, in_f), lambda s: (0, 0)),
            pl.BlockSpec((1, grid_s * TS), lambda s: (0, 0)),
            pl.BlockSpec(memory_space=pltpu.MemorySpace.HBM),
        ],
        out_specs=pl.BlockSpec((batch, TS), lambda s: (0, s)),
        out_shape=jax.ShapeDtypeStruct((batch, out_f), jnp.float32),
        scratch_shapes=[
            pltpu.VMEM((2, SLAB, IN_F), jnp.float32),
            pltpu.SemaphoreType.DMA((2,)),
        ],
    )(x, b2, W)


# speedup vs baseline: 1.0288x; 1.0288x over previous
import jax
import jax.numpy as jnp
from jax.experimental import pallas as pl
from jax.experimental.pallas import tpu as pltpu

IN_F = 1024
TS = 1024
R = 4
SLAB = TS * R


def _make_body(out_f, nslab, last_rows, grid_s):
    def body(x_ref, b_ref, w_hbm, o_ref, w_buf, sem):
        s = pl.program_id(0)
        j = s // R
        i = s % R
        k = j % 2

        def full_copy(jn, kn):
            return pltpu.make_async_copy(
                w_hbm.at[pl.ds(jn * SLAB, SLAB), :], w_buf.at[kn],
                sem.at[kn])

        def short_copy(jn, kn):
            return pltpu.make_async_copy(
                w_hbm.at[pl.ds(jn * SLAB, last_rows), :],
                w_buf.at[kn, pl.ds(0, last_rows), :], sem.at[kn])

        def start_slab(jn, kn):
            @pl.when(jn < nslab - 1)
            def _():
                full_copy(jn, kn).start()

            @pl.when(jn == nslab - 1)
            def _():
                short_copy(jn, kn).start()

        @pl.when(i == 0)
        def _fetch():
            @pl.when(s == 0)
            def _prologue():
                start_slab(0, 0)

            @pl.when(j + 1 < nslab)
            def _next():
                start_slab(j + 1, (j + 1) % 2)

            @pl.when(j < nslab - 1)
            def _():
                full_copy(j, k).wait()

            @pl.when(j == nslab - 1)
            def _():
                short_copy(j, k).wait()

        w = w_buf[k, pl.ds(i * TS, TS), :]
        acc = jax.lax.dot_general(
            x_ref[...], w,
            dimension_numbers=(((1,), (1,)), ((), ())),
            preferred_element_type=jnp.float32,
            precision=jax.lax.Precision.DEFAULT,
        )
        o_ref[...] = acc + b_ref[:, pl.ds(s * TS, TS)]

    return body


def kernel(x, W, b):
    batch, in_f = x.shape
    out_f = W.shape[0]
    grid_s = (out_f + TS - 1) // TS
    nslab = (out_f + SLAB - 1) // SLAB
    last_rows = out_f - (nslab - 1) * SLAB
    b2 = jnp.pad(b, (0, grid_s * TS - out_f)).reshape(1, grid_s * TS)
    return pl.pallas_call(
        _make_body(out_f, nslab, last_rows, grid_s),
        grid=(grid_s,),
        in_specs=[
            pl.BlockSpec((batch, in_f), lambda s: (0, 0)),
            pl.BlockSpec((1, grid_s * TS), lambda s: (0, 0)),
            pl.BlockSpec(memory_space=pltpu.MemorySpace.HBM),
        ],
        out_specs=pl.BlockSpec((batch, TS), lambda s: (0, s)),
        out_shape=jax.ShapeDtypeStruct((batch, out_f), jnp.float32),
        scratch_shapes=[
            pltpu.VMEM((2, SLAB, IN_F), jnp.float32),
            pltpu.SemaphoreType.DMA((2,)),
        ],
    )(x, b2, W)
